# baseline (speedup 1.0000x reference)
import functools

import jax
import jax.numpy as jnp
from jax import lax
from jax.experimental import pallas as pl
from jax.experimental.pallas import tpu as pltpu
from jax.experimental.pallas import tpu_sc as plsc

NC = 2
NS = 16
NW = NC * NS
LANES = 16
CH = 2
SCH = 16
GID = 64


def _sc_gather_reduce(nodes, nlflat, edpack, n_pad, K, F, DE):
  N = nodes.shape[0]
  n_per_w = n_pad // NW
  supers = n_per_w // SCH
  subs = SCH // CH
  dpc = CH * K // GID
  fchunks = F // LANES
  jstep = LANES // DE
  rows_stage = N // NS
  mesh = plsc.VectorSubcoreMesh(core_axis_name="c", subcore_axis_name="s")

  @functools.partial(
      pl.kernel,
      out_type=jax.ShapeDtypeStruct((n_pad, DE, F), jnp.float32),
      mesh=mesh,
      scratch_types=[
          pltpu.VMEM((SCH * K,), jnp.int32),
          pltpu.VMEM((SCH, K * DE), jnp.float32),
          pltpu.VMEM((2, CH * K, F), jnp.float32),
          pltpu.VMEM((SCH, DE, F), jnp.float32),
          pltpu.VMEM_SHARED((N, F), jnp.float32),
          pltpu.SemaphoreType.DMA,
          pltpu.SemaphoreType.DMA,
      ],
  )
  def sc_kernel(nodes_hbm, nl_hbm, ed_hbm, t_hbm, nl_v, ed_v, rows_v,
                out_v, ns_v, sem0, sem1):
    wid = lax.axis_index("s") * NC + lax.axis_index("c")
    base = wid * n_per_w
    sems = (sem0, sem1)

    sid = lax.axis_index("s")

    def stage_body(it, carry):
      r0 = sid * rows_stage + it * 32
      pltpu.sync_copy(nodes_hbm.at[pl.ds(r0, 32)], ns_v.at[pl.ds(r0, 32)])
      return carry

    lax.fori_loop(0, rows_stage // 32, stage_body, 0)
    plsc.subcore_barrier()

    def fire(sub, buf):
      return [
          pltpu.async_copy(
              ns_v.at[nl_v.at[pl.ds((sub * dpc + h) * GID, GID)]],
              rows_v.at[buf, pl.ds(h * GID, GID)],
              sems[buf],
          )
          for h in range(dpc)
      ]

    def compute_node(sub, buf, c):
      FCB = 4
      zero = jnp.zeros((LANES,), jnp.float32)
      for blk in range(fchunks // FCB):
        acc0 = tuple(zero for _ in range(DE * FCB))

        def j_body(j4, acc, blk=blk):
          jbase = c * K + j4 * jstep
          ev = ed_v[sub * CH + c, pl.ds(j4 * LANES, LANES)]
          new = list(acc)
          for jj in range(jstep):
            r = [
                rows_v[buf, jbase + jj,
                       pl.ds((blk * FCB + fc) * LANES, LANES)]
                for fc in range(FCB)
            ]
            for n in range(DE):
              e = ev[jj * DE + n]
              for fc in range(FCB):
                new[n * FCB + fc] = new[n * FCB + fc] + e * r[fc]
          return tuple(new)

        acc = lax.fori_loop(0, K // jstep, j_body, acc0, unroll=2)
        for n in range(DE):
          for fc in range(FCB):
            out_v[sub * CH + c, n, pl.ds((blk * FCB + fc) * LANES, LANES)] = (
                acc[n * FCB + fc])

    def super_body(s, carry):
      row0 = base + s * SCH
      pltpu.sync_copy(nl_hbm.at[pl.ds(row0 * K, SCH * K)], nl_v)
      pltpu.sync_copy(ed_hbm.at[pl.ds(row0, SCH)], ed_v)
      pending = [fire(0, 0), fire(1, 1)]
      for sub in range(subs):
        buf = sub % 2
        for cp in pending[buf]:
          cp.wait()
        lax.fori_loop(
            0, CH,
            lambda c, u, sub=sub, buf=buf: compute_node(sub, buf, c) or u,
            0)
        if sub + 2 < subs:
          pending[buf] = fire(sub + 2, buf)
      pltpu.sync_copy(out_v, t_hbm.at[pl.ds(row0, SCH)])
      return carry

    lax.fori_loop(0, supers, super_body, 0)

  return sc_kernel(nodes, nlflat, edpack)


def _tc_matmul(t2, w2, inv_p, n_pad, F, DE):
  BM = 256
  grid = n_pad // BM

  def body(t_ref, w_ref, inv_ref, o_ref):
    o_ref[...] = jnp.dot(
        t_ref[...], w_ref[...], preferred_element_type=jnp.float32
    ) * inv_ref[...]

  return pl.pallas_call(
      body,
      grid=(grid,),
      in_specs=[
          pl.BlockSpec((BM, DE * F), lambda i: (i, 0)),
          pl.BlockSpec((DE * F, F), lambda i: (0, 0)),
          pl.BlockSpec((BM, 1), lambda i: (i, 0)),
      ],
      out_specs=pl.BlockSpec((BM, F), lambda i: (i, 0)),
      out_shape=jax.ShapeDtypeStruct((n_pad, F), jnp.float32),
  )(t2, w2, inv_p)


def kernel(nodes, nlist, edges, inv_degree, w):
  N, F = nodes.shape
  K = nlist.shape[1]
  DE = edges.shape[2]
  block = NW * SCH
  n_pad = ((N + block - 1) // block) * block
  pad = n_pad - N

  nstage = NS * 32
  nodes_p = jnp.pad(nodes, ((0, (-N) % nstage), (0, 0)))
  nlflat = jnp.pad(nlist.astype(jnp.int32), ((0, pad), (0, 0))).reshape(-1)
  edpack = jnp.pad(edges, ((0, pad), (0, 0), (0, 0))).reshape(n_pad, K * DE)
  inv_p = jnp.pad(inv_degree, (0, pad)).reshape(n_pad, 1)

  t = _sc_gather_reduce(nodes_p, nlflat, edpack, n_pad, K, F, DE)
  t2 = t.reshape(n_pad, DE * F)
  w2 = w.transpose(2, 0, 1).reshape(DE * F, F)
  out = _tc_matmul(t2, w2, inv_p, n_pad, F, DE)
  return out[:N]


# speedup vs baseline: 1.4187x; 1.4187x over previous
import functools

import jax
import jax.numpy as jnp
from jax import lax
from jax.experimental import pallas as pl
from jax.experimental.pallas import tpu as pltpu
from jax.experimental.pallas import tpu_sc as plsc

NC = 2
NS = 16
NW = NC * NS
LANES = 16
CH = 2
SCH = 16
GID = 64


def _sc_gather_reduce(nodes, nlflat, edpack, n_pad, K, F, DE):
  N = nodes.shape[0]
  n_per_w = n_pad // NW
  supers = n_per_w // SCH
  subs = SCH // CH
  dpc = CH * K // GID
  fchunks = F // LANES
  jstep = LANES // DE
  rows_stage = N // NS
  mesh = plsc.VectorSubcoreMesh(core_axis_name="c", subcore_axis_name="s")

  @functools.partial(
      pl.kernel,
      out_type=jax.ShapeDtypeStruct((n_pad, DE, F), jnp.float32),
      mesh=mesh,
      scratch_types=[
          pltpu.VMEM((SCH * K,), jnp.int32),
          pltpu.VMEM((SCH, K * DE), jnp.float32),
          pltpu.VMEM((2, CH * K, F), jnp.float32),
          pltpu.VMEM((SCH, DE, F), jnp.float32),
          pltpu.VMEM_SHARED((N, F), jnp.float32),
          pltpu.SemaphoreType.DMA,
          pltpu.SemaphoreType.DMA,
      ],
  )
  def sc_kernel(nodes_hbm, nl_hbm, ed_hbm, t_hbm, nl_v, ed_v, rows_v,
                out_v, ns_v, sem0, sem1):
    wid = lax.axis_index("s") * NC + lax.axis_index("c")
    base = wid * n_per_w
    sems = (sem0, sem1)

    sid = lax.axis_index("s")

    def stage_body(it, carry):
      r0 = sid * rows_stage + it * 32
      pltpu.sync_copy(nodes_hbm.at[pl.ds(r0, 32)], ns_v.at[pl.ds(r0, 32)])
      return carry

    lax.fori_loop(0, rows_stage // 32, stage_body, 0)
    plsc.subcore_barrier()

    def fire(sub, buf):
      return [
          pltpu.async_copy(
              ns_v.at[nl_v.at[pl.ds((sub * dpc + h) * GID, GID)]],
              rows_v.at[buf, pl.ds(h * GID, GID)],
              sems[buf],
          )
          for h in range(dpc)
      ]

    def compute_node(sub, buf, c):
      FCB = 4
      zero = jnp.zeros((LANES,), jnp.float32)
      for blk in range(fchunks // FCB):
        acc0 = tuple(zero for _ in range(DE * FCB))

        def j_body(j4, acc, blk=blk):
          jbase = c * K + j4 * jstep
          ev = ed_v[sub * CH + c, pl.ds(j4 * LANES, LANES)]
          new = list(acc)
          for jj in range(jstep):
            r = [
                rows_v[buf, jbase + jj,
                       pl.ds((blk * FCB + fc) * LANES, LANES)]
                for fc in range(FCB)
            ]
            for n in range(DE):
              e = ev[jj * DE + n]
              for fc in range(FCB):
                new[n * FCB + fc] = new[n * FCB + fc] + e * r[fc]
          return tuple(new)

        acc = lax.fori_loop(0, K // jstep, j_body, acc0, unroll=4)
        for n in range(DE):
          for fc in range(FCB):
            out_v[sub * CH + c, n, pl.ds((blk * FCB + fc) * LANES, LANES)] = (
                acc[n * FCB + fc])

    def super_body(s, carry):
      row0 = base + s * SCH
      pltpu.sync_copy(nl_hbm.at[pl.ds(row0 * K, SCH * K)], nl_v)
      pltpu.sync_copy(ed_hbm.at[pl.ds(row0, SCH)], ed_v)
      pending = [fire(0, 0), fire(1, 1)]
      for sub in range(subs):
        buf = sub % 2
        for cp in pending[buf]:
          cp.wait()
        lax.fori_loop(
            0, CH,
            lambda c, u, sub=sub, buf=buf: compute_node(sub, buf, c) or u,
            0)
        if sub + 2 < subs:
          pending[buf] = fire(sub + 2, buf)
      pltpu.sync_copy(out_v, t_hbm.at[pl.ds(row0, SCH)])
      return carry

    lax.fori_loop(0, supers, super_body, 0)

  return sc_kernel(nodes, nlflat, edpack)


def _tc_matmul(t2, w2, inv_p, n_pad, F, DE):
  BM = 256
  grid = n_pad // BM

  def body(t_ref, w_ref, inv_ref, o_ref):
    o_ref[...] = jnp.dot(
        t_ref[...], w_ref[...], preferred_element_type=jnp.float32
    ) * inv_ref[...]

  return pl.pallas_call(
      body,
      grid=(grid,),
      in_specs=[
          pl.BlockSpec((BM, DE * F), lambda i: (i, 0)),
          pl.BlockSpec((DE * F, F), lambda i: (0, 0)),
          pl.BlockSpec((BM, 1), lambda i: (i, 0)),
      ],
      out_specs=pl.BlockSpec((BM, F), lambda i: (i, 0)),
      out_shape=jax.ShapeDtypeStruct((n_pad, F), jnp.float32),
  )(t2, w2, inv_p)


def kernel(nodes, nlist, edges, inv_degree, w):
  N, F = nodes.shape
  K = nlist.shape[1]
  DE = edges.shape[2]
  block = NW * SCH
  n_pad = ((N + block - 1) // block) * block
  pad = n_pad - N

  nstage = NS * 32
  nodes_p = jnp.pad(nodes, ((0, (-N) % nstage), (0, 0)))
  nlflat = jnp.pad(nlist.astype(jnp.int32), ((0, pad), (0, 0))).reshape(-1)
  edpack = jnp.pad(edges, ((0, pad), (0, 0), (0, 0))).reshape(n_pad, K * DE)
  inv_p = jnp.pad(inv_degree, (0, pad)).reshape(n_pad, 1)

  t = _sc_gather_reduce(nodes_p, nlflat, edpack, n_pad, K, F, DE)
  t2 = t.reshape(n_pad, DE * F)
  w2 = w.transpose(2, 0, 1).reshape(DE * F, F)
  out = _tc_matmul(t2, w2, inv_p, n_pad, F, DE)
  return out[:N]
